# baseline (speedup 1.0000x reference)
import functools

import jax
import jax.numpy as jnp
from jax import lax
from jax.experimental import pallas as pl
from jax.experimental.pallas import tpu as pltpu
from jax.experimental.pallas import tpu_sc as plsc

VOCAB = 100000
EMB_DIM = 128
B, L = 4096, 200

GRID_T = 10
ROWS_T = VOCAB // GRID_T
TOK_B = (B * L) // GRID_T

NC, NS = 2, 16
NW = NC * NS
TOK_W = (B * L) // NW
ROWS_W = B // NW
GROUPS_W = ROWS_W // 16
UNROLL = 8
T_PAD = VOCAB + 16


def _tc_body(w1_ref, w2_ref, b1_ref, b2_ref, tbl_ref, x_ref, m_ref,
             t_ref, xm_ref, c_ref):
    v = jnp.dot(w2_ref[...], w1_ref[...], preferred_element_type=jnp.float32)
    t_ref[0, :, :] = lax.dot_general(
        v, tbl_ref[...], (((1,), (1,)), ((), ())),
        preferred_element_type=jnp.float32,
    )
    xm_ref[0, 0, :] = jnp.where(m_ref[0, 0, :] != 0, x_ref[0, 0, :], VOCAB)
    c = jnp.sum(w2_ref[...] * b1_ref[...]) + b2_ref[0, 0]
    c_ref[...] = jnp.full((1, 128), c, jnp.float32)


def _tc_stage(emb_table, W1, b1, W2, b2, x, mask):
    x3 = x.astype(jnp.int32).reshape(GRID_T, 1, TOK_B)
    m3 = mask.reshape(GRID_T, 1, TOK_B)
    t3, xm3, c_out = pl.pallas_call(
        _tc_body,
        grid=(GRID_T,),
        in_specs=[
            pl.BlockSpec((64, EMB_DIM), lambda i: (0, 0)),
            pl.BlockSpec((1, 64), lambda i: (0, 0)),
            pl.BlockSpec((1, 64), lambda i: (0, 0)),
            pl.BlockSpec((1, 1), lambda i: (0, 0)),
            pl.BlockSpec((ROWS_T, EMB_DIM), lambda i: (i, 0)),
            pl.BlockSpec((1, 1, TOK_B), lambda i: (i, 0, 0)),
            pl.BlockSpec((1, 1, TOK_B), lambda i: (i, 0, 0)),
        ],
        out_specs=[
            pl.BlockSpec((1, 1, ROWS_T), lambda i: (i, 0, 0)),
            pl.BlockSpec((1, 1, TOK_B), lambda i: (i, 0, 0)),
            pl.BlockSpec((1, 128), lambda i: (0, 0)),
        ],
        out_shape=[
            jax.ShapeDtypeStruct((GRID_T, 1, ROWS_T), jnp.float32),
            jax.ShapeDtypeStruct((GRID_T, 1, TOK_B), jnp.int32),
            jax.ShapeDtypeStruct((1, 128), jnp.float32),
        ],
    )(W1, W2, b1.reshape(1, 64), b2.reshape(1, 1), emb_table, x3, m3)
    return t3.reshape(VOCAB), xm3.reshape(B * L), c_out.reshape(128)


_SC_MESH = plsc.VectorSubcoreMesh(
    core_axis_name="c", subcore_axis_name="s", num_cores=NC, num_subcores=NS
)


@functools.partial(
    pl.kernel,
    out_type=jax.ShapeDtypeStruct((B,), jnp.float32),
    mesh=_SC_MESH,
    compiler_params=pltpu.CompilerParams(needs_layout_passes=False),
    scratch_types=[
        pltpu.VMEM((T_PAD,), jnp.float32),
        pltpu.VMEM((TOK_W,), jnp.int32),
        pltpu.VMEM((ROWS_W,), jnp.float32),
        pltpu.VMEM((16,), jnp.float32),
        pltpu.SemaphoreType.DMA,
        pltpu.SemaphoreType.DMA,
        pltpu.SemaphoreType.DMA,
    ],
)
def _sc_pool(t_hbm, xm_hbm, c_hbm, out_hbm, t_v, idx_v, o_v, c_v,
             sem_t, sem_i, sem_c):
    wid = lax.axis_index("s") * NC + lax.axis_index("c")
    zero16 = jnp.zeros((16,), jnp.float32)
    one16 = jnp.ones((16,), jnp.float32)
    with jax.named_scope("sc_copy"):
        ct = pltpu.async_copy(t_hbm, t_v.at[pl.ds(0, VOCAB)], sem_t)
        ci = pltpu.async_copy(
            xm_hbm.at[pl.ds(wid * TOK_W, TOK_W)], idx_v, sem_i)
        cc = pltpu.async_copy(c_hbm.at[pl.ds(0, 16)], c_v, sem_c)
        t_v[pl.ds(VOCAB, 16)] = zero16
        ct.wait()
        ci.wait()
        cc.wait()

    lanes = lax.broadcasted_iota(jnp.int32, (16,), 0)
    c16 = c_v[...]
    sink16 = jnp.full((16,), VOCAB, jnp.int32)

    with jax.named_scope("sc_compute"):
        for g in range(GROUPS_W):
            goff16 = (g * 16 + lanes) * L

            def body(kk, carry, goff16=goff16):
                acc, mac = carry
                for u in range(UNROLL):
                    xi = plsc.load_gather(idx_v, [goff16 + (kk * UNROLL + u)])
                    acc = acc + plsc.load_gather(t_v, [xi])
                    mac = mac + jnp.where(xi < sink16, one16, zero16)
                return acc, mac

            acc, mac = lax.fori_loop(0, L // UNROLL, body, (zero16, zero16))
            z = acc / jnp.maximum(mac, one16) + c16
            o_v[pl.ds(g * 16, 16)] = one16 / (one16 + jnp.exp(-z))

    pltpu.sync_copy(o_v, out_hbm.at[pl.ds(wid * ROWS_W, ROWS_W)])


def kernel(x, attention_mask, emb_table, W1, b1, W2, b2):
    t, xm, c_vec = _tc_stage(emb_table, W1, b1, W2, b2, x, attention_mask)
    return _sc_pool(t, xm, c_vec)


# speedup vs baseline: 1.0303x; 1.0303x over previous
import functools

import jax
import jax.numpy as jnp
from jax import lax
from jax.experimental import pallas as pl
from jax.experimental.pallas import tpu as pltpu
from jax.experimental.pallas import tpu_sc as plsc

VOCAB = 100000
EMB_DIM = 128
B, L = 4096, 200

GRID_T = 10
ROWS_T = VOCAB // GRID_T

NC, NS = 2, 16
NW = NC * NS
ROWS_W = B // NW
GROUPS_W = ROWS_W // 16
UNROLL = 8


def _tc_body(w1_ref, w2_ref, b1_ref, b2_ref, tbl_ref, t_ref, c_ref):
    v = jnp.dot(w2_ref[...], w1_ref[...], preferred_element_type=jnp.float32)
    t_ref[0, :, :] = lax.dot_general(
        v, tbl_ref[...], (((1,), (1,)), ((), ())),
        preferred_element_type=jnp.float32,
    )
    c = jnp.sum(w2_ref[...] * b1_ref[...]) + b2_ref[0, 0]
    c_ref[...] = jnp.full((1, 128), c, jnp.float32)


def _tc_stage(emb_table, W1, b1, W2, b2):
    t3, c_out = pl.pallas_call(
        _tc_body,
        grid=(GRID_T,),
        in_specs=[
            pl.BlockSpec((64, EMB_DIM), lambda i: (0, 0)),
            pl.BlockSpec((1, 64), lambda i: (0, 0)),
            pl.BlockSpec((1, 64), lambda i: (0, 0)),
            pl.BlockSpec((1, 1), lambda i: (0, 0)),
            pl.BlockSpec((ROWS_T, EMB_DIM), lambda i: (i, 0)),
        ],
        out_specs=[
            pl.BlockSpec((1, 1, ROWS_T), lambda i: (i, 0, 0)),
            pl.BlockSpec((1, 128), lambda i: (0, 0)),
        ],
        out_shape=[
            jax.ShapeDtypeStruct((GRID_T, 1, ROWS_T), jnp.float32),
            jax.ShapeDtypeStruct((1, 128), jnp.float32),
        ],
    )(W1, W2, b1.reshape(1, 64), b2.reshape(1, 1), emb_table)
    return t3.reshape(VOCAB), c_out.reshape(128)


_SC_MESH = plsc.VectorSubcoreMesh(
    core_axis_name="c", subcore_axis_name="s", num_cores=NC, num_subcores=NS
)


@functools.partial(
    pl.kernel,
    out_type=jax.ShapeDtypeStruct((B,), jnp.float32),
    mesh=_SC_MESH,
    compiler_params=pltpu.CompilerParams(needs_layout_passes=False),
    scratch_types=[
        pltpu.VMEM((VOCAB,), jnp.float32),
        pltpu.VMEM((16, L), jnp.int32),
        pltpu.VMEM((16, L), jnp.int32),
        pltpu.VMEM((16, L), jnp.int32),
        pltpu.VMEM((16, L), jnp.int32),
        pltpu.VMEM((ROWS_W,), jnp.float32),
        pltpu.VMEM((16,), jnp.float32),
        pltpu.SemaphoreType.DMA,
        pltpu.SemaphoreType.DMA,
        pltpu.SemaphoreType.DMA,
        pltpu.SemaphoreType.DMA,
        pltpu.SemaphoreType.DMA,
        pltpu.SemaphoreType.DMA,
    ],
)
def _sc_pool(t_hbm, x_hbm, m_hbm, c_hbm, out_hbm,
             t_v, xw0, xw1, mw0, mw1, o_v, c_v,
             sem_t, sem_c, sem_x0, sem_x1, sem_m0, sem_m1):
    wid = lax.axis_index("s") * NC + lax.axis_index("c")
    row0 = wid * ROWS_W
    zero16 = jnp.zeros((16,), jnp.float32)
    one16 = jnp.ones((16,), jnp.float32)
    lanes = lax.broadcasted_iota(jnp.int32, (16,), 0)

    xw = [xw0, xw1]
    mw = [mw0, mw1]
    sx = [sem_x0, sem_x1]
    sm = [sem_m0, sem_m1]
    dx = [None, None]
    dm = [None, None]

    ct = pltpu.async_copy(t_hbm, t_v, sem_t)
    cc = pltpu.async_copy(c_hbm.at[pl.ds(0, 16)], c_v, sem_c)
    dx[0] = pltpu.async_copy(x_hbm.at[pl.ds(row0, 16)], xw[0], sem_x0)
    dm[0] = pltpu.async_copy(m_hbm.at[pl.ds(row0, 16)], mw[0], sem_m0)
    ct.wait()
    cc.wait()
    c16 = c_v[...]

    for g in range(GROUPS_W):
        p = g % 2
        if g + 1 < GROUPS_W:
            q = (g + 1) % 2
            dx[q] = pltpu.async_copy(
                x_hbm.at[pl.ds(row0 + (g + 1) * 16, 16)], xw[q], sx[q])
            dm[q] = pltpu.async_copy(
                m_hbm.at[pl.ds(row0 + (g + 1) * 16, 16)], mw[q], sm[q])
        dx[p].wait()
        dm[p].wait()
        xwin, mwin = xw[p], mw[p]

        def body(kk, carry, xwin=xwin, mwin=mwin):
            acc, mac = carry
            for u in range(UNROLL):
                kv = jnp.full((16,), kk * UNROLL + u, jnp.int32)
                xi = plsc.load_gather(xwin, [lanes, kv])
                mi = plsc.load_gather(mwin, [lanes, kv])
                mf = mi.astype(jnp.float32)
                acc = acc + plsc.load_gather(t_v, [xi]) * mf
                mac = mac + mf
            return acc, mac

        acc, mac = lax.fori_loop(0, L // UNROLL, body, (zero16, zero16))
        z = acc / jnp.maximum(mac, one16) + c16
        o_v[pl.ds(g * 16, 16)] = one16 / (one16 + jnp.exp(-z))

    pltpu.sync_copy(o_v, out_hbm.at[pl.ds(row0, ROWS_W)])


def kernel(x, attention_mask, emb_table, W1, b1, W2, b2):
    t, c_vec = _tc_stage(emb_table, W1, b1, W2, b2)
    return _sc_pool(t, x.astype(jnp.int32), attention_mask, c_vec)
